# Initial kernel scaffold; baseline (speedup 1.0000x reference)
import jax
import jax.numpy as jnp
from jax.experimental import pallas as pl


def kernel(encoded_parent, log_kmer_rates, log_site_rates):
    raise NotImplementedError("write your pallas kernel here")


# speedup vs baseline: 1.4169x; 1.4169x over previous
import functools

import jax
import jax.numpy as jnp
from jax import lax
from jax.experimental import pallas as pl
from jax.experimental.pallas import tpu as pltpu
from jax.experimental.pallas import tpu_sc as plsc

SEQ_LEN = 8192
NUM_CORES = 2
NUM_SUBCORES = 16
LANES = 16
NUM_WORKERS = NUM_CORES * NUM_SUBCORES
BPW = SEQ_LEN // NUM_WORKERS

_mesh = plsc.VectorSubcoreMesh(core_axis_name="c", subcore_axis_name="s")


@functools.partial(
    pl.kernel,
    mesh=_mesh,
    out_type=jax.ShapeDtypeStruct((SEQ_LEN,), jnp.float32),
    scratch_types=[
        pltpu.VMEM((BPW,), jnp.int32),
        pltpu.VMEM((BPW,), jnp.float32),
        pltpu.VMEM((BPW,), jnp.float32),
        pltpu.VMEM((BPW,), jnp.float32),
        pltpu.SemaphoreType.DMA,
    ],
)
def _shmoof_sc(idx_hbm, kmer_hbm, site_hbm, out_hbm, idx_v, lk_v, ls_v, out_v, sem):
    wid = lax.axis_index("s") * NUM_CORES + lax.axis_index("c")
    base = wid * BPW
    pltpu.sync_copy(idx_hbm.at[pl.ds(base, BPW)], idx_v)
    gather = pltpu.async_copy(kmer_hbm.at[idx_v], lk_v, sem)
    pltpu.sync_copy(site_hbm.at[pl.ds(base, BPW)], ls_v)
    gather.wait()
    for i in range(BPW // LANES):
        sl = pl.ds(i * LANES, LANES)
        out_v[sl] = jnp.exp(lk_v[sl] + ls_v[sl])
    pltpu.sync_copy(out_v, out_hbm.at[pl.ds(base, BPW)])


def kernel(encoded_parent, log_kmer_rates, log_site_rates):
    return _shmoof_sc(
        encoded_parent,
        log_kmer_rates.reshape(-1),
        log_site_rates.reshape(-1)[:SEQ_LEN],
    )
